# baseline (speedup 1.0000x reference)
import functools

import jax
import jax.numpy as jnp
from jax import lax
from jax.experimental import pallas as pl
from jax.experimental.pallas import tpu as pltpu
from jax.experimental.pallas import tpu_sc as plsc

L = 16
NC = 2
NS = 16
NW = NC * NS
CB = 128
BLK = 1024
NTC = 68


def _tc_part(features, state_row, batch3, HT):
    G = state_row.shape[1]

    def body(feat_ref, b_ref, s_ref, e_ref, x_ref, y_ref, z_ref, eye_ref):
        b = pl.program_id(0)

        @pl.when(b == 0)
        def _():
            r = lax.broadcasted_iota(jnp.int32, (BLK, BLK), 0)
            c = lax.broadcasted_iota(jnp.int32, (BLK, BLK), 1)
            eye_ref[...] = (r == c).astype(jnp.float32)

        bc = b_ref[...].reshape(1, BLK)
        gids = lax.broadcasted_iota(jnp.int32, (G, BLK), 0)
        onehot_t = (bc == gids).astype(jnp.float32)
        s = lax.dot_general(onehot_t, s_ref[...],
                            (((0,), (1,)), ((), ())),
                            precision=lax.Precision.HIGHEST,
                            preferred_element_type=jnp.float32)
        F = feat_ref[...]
        e = jnp.where(s == 0., F[:, 0:1],
                      jnp.where(s == 1., F[:, 1:2], F[:, 2:3]))
        cols = jnp.concatenate([e, F[:, 4:5], F[:, 2:3], F[:, 3:4]], axis=1)
        rows = lax.dot_general(cols, eye_ref[...],
                               (((0,), (0,)), ((), ())),
                               precision=lax.Precision.HIGHEST,
                               preferred_element_type=jnp.float32)
        e_ref[...] = rows[0:1, :].reshape(1, 1, BLK)
        x_ref[...] = rows[1:2, :].reshape(1, 1, BLK)
        y_ref[...] = rows[2:3, :].reshape(1, 1, BLK)
        z_ref[...] = rows[3:4, :].reshape(1, 1, BLK)

    row = jax.ShapeDtypeStruct((NTC, 1, BLK), jnp.float32)
    outs = pl.pallas_call(
        body,
        grid=(NTC,),
        in_specs=[
            pl.BlockSpec((BLK, CB), lambda i: (i, 0)),
            pl.BlockSpec((1, 1, BLK), lambda i: (i, 0, 0)),
            pl.BlockSpec((1, G), lambda i: (0, 0)),
        ],
        out_specs=[pl.BlockSpec((1, 1, BLK), lambda i: (i, 0, 0))] * 4,
        out_shape=[row] * 4,
        scratch_shapes=[pltpu.VMEM((BLK, BLK), jnp.float32)],
        compiler_params=pltpu.CompilerParams(
            dimension_semantics=("arbitrary",)),
    )(features, batch3, state_row)

    return tuple(a.reshape(HT) for a in outs)


def _sc_part(features, state_flat, batch, LO, N):
    G = state_flat.shape[0]
    NSC = N - LO
    CPW = ((NSC + NW - 1) // NW + L - 1) // L * L
    C = 96
    NCHUNK = CPW // C
    NBUF = 5
    assert CPW % C == 0 and NCHUNK % NBUF == 0 and C % L == 0

    mesh = plsc.VectorSubcoreMesh(core_axis_name="c", subcore_axis_name="s")

    @functools.partial(
        pl.kernel,
        mesh=mesh,
        compiler_params=pltpu.CompilerParams(
            needs_layout_passes=False,
            skip_device_barrier=True,
            disable_bounds_checks=True,
            disable_semaphore_checks=True,
        ),
        out_type=tuple(
            jax.ShapeDtypeStruct((NSC,), jnp.float32) for _ in range(4)),
        scratch_types=(
            [pltpu.VMEM((C, CB), jnp.float32)] * NBUF
            + [
                pltpu.VMEM((CPW,), jnp.int32),
                pltpu.VMEM((G,), jnp.int32),
                pltpu.VMEM((CPW,), jnp.float32),
                pltpu.VMEM((CPW,), jnp.float32),
                pltpu.VMEM((CPW,), jnp.float32),
                pltpu.VMEM((CPW,), jnp.float32),
            ]
            + [pltpu.SemaphoreType.DMA] * (NBUF + 2)
        ),
    )
    def sc_kernel(feat_hbm, state_hbm, batch_hbm, e_hbm, x_hbm, y_hbm, z_hbm,
                  *scratch):
        fbufs = scratch[:NBUF]
        batch_v, state_v, e_v, x_v, y_v, z_v = scratch[NBUF:NBUF + 6]
        sems = scratch[NBUF + 6:NBUF + 6 + NBUF]
        sem_in, sem_out = scratch[NBUF + 6 + NBUF:]
        wid = lax.axis_index("s") * NC + lax.axis_index("c")
        base = LO + jnp.minimum(wid * CPW, NSC - CPW)

        def feat_dma(t, b):
            return pltpu.make_async_copy(
                feat_hbm.at[pl.ds(base + t * C, C), pl.ds(0, CB)],
                fbufs[b], sems[b])

        cp_state = pltpu.make_async_copy(state_hbm, state_v, sem_in)
        cp_batch = pltpu.make_async_copy(
            batch_hbm.at[pl.ds(base - LO, CPW)], batch_v, sem_in)
        cp_state.start()
        cp_batch.start()
        for t in range(NBUF - 1):
            feat_dma(t, t).start()
        cp_state.wait()
        cp_batch.wait()

        iota = lax.iota(jnp.int32, L)
        c2 = jnp.full((L,), 2, jnp.int32)
        c3 = jnp.full((L,), 3, jnp.int32)
        c4 = jnp.full((L,), 4, jnp.int32)

        def outer(k, carry):
            t0 = k * NBUF
            for b in range(NBUF):
                t = t0 + b
                nxt = t + NBUF - 1

                @pl.when(nxt < NCHUNK)
                def _(nxt=nxt, b=b):
                    feat_dma(nxt, (b + NBUF - 1) % NBUF).start()

                feat_dma(t, b).wait()
                fb = fbufs[b]

                def body(j, carry2, t=t, fb=fb):
                    n = iota + j * L
                    g = batch_v[pl.ds(t * C + j * L, L)]
                    s = plsc.load_gather(state_v, [g])
                    e = plsc.load_gather(fb, [n, s])
                    x = plsc.load_gather(fb, [n, c4])
                    y = plsc.load_gather(fb, [n, c2])
                    z = plsc.load_gather(fb, [n, c3])
                    o = pl.ds(t * C + j * L, L)
                    e_v[o] = e
                    x_v[o] = x
                    y_v[o] = y
                    z_v[o] = z
                    return carry2

                lax.fori_loop(0, C // L, body, 0)
            return carry

        lax.fori_loop(0, NCHUNK // NBUF, outer, 0)

        outs = [
            pltpu.make_async_copy(v, h.at[pl.ds(base - LO, CPW)], sem_out)
            for v, h in ((e_v, e_hbm), (x_v, x_hbm), (y_v, y_hbm), (z_v, z_hbm))
        ]
        for cp in outs:
            cp.start()
        for cp in outs:
            cp.wait()

    batch_tail = lax.slice_in_dim(batch, LO, N)
    return sc_kernel(features, state_flat, batch_tail)


def kernel(features, state, batch):
    N, D = features.shape
    G = state.shape[0]
    HT = NTC * BLK

    state_flat = state.reshape(G)
    state_row = state.reshape(1, G).astype(jnp.float32)
    batch3 = batch[:HT].reshape(NTC, 1, BLK)

    e_t, x_t, y_t, z_t = _tc_part(features, state_row, batch3, HT)
    e_s, x_s, y_s, z_s = _sc_part(features, state_flat, batch, HT, N)

    e = jnp.concatenate([e_t, e_s])
    x = jnp.concatenate([x_t, x_s])
    y = jnp.concatenate([y_t, y_s])
    z = jnp.concatenate([z_t, z_s])
    return e.reshape(N, 1), jnp.stack([x, y, z], axis=-1)


# speedup vs baseline: 6.3774x; 6.3774x over previous
import functools

import jax
import jax.numpy as jnp
from jax import lax
from jax.experimental import pallas as pl
from jax.experimental.pallas import tpu as pltpu
from jax.experimental.pallas import tpu_sc as plsc

L = 16
NC = 2
NS = 16
NW = NC * NS
CB = 128
BLK = 1024
NTC = 68


def _tc_part(features, state_row, batch3, HT):
    G = state_row.shape[1]

    def body(feat_ref, b_ref, s_ref, e_ref, x_ref, y_ref, z_ref):
        bc = b_ref[...].reshape(1, BLK)
        gids = lax.broadcasted_iota(jnp.int32, (G, BLK), 0)
        onehot_t = (bc == gids).astype(jnp.float32)
        s_row = lax.dot_general(s_ref[...], onehot_t,
                                (((1,), (0,)), ((), ())),
                                preferred_element_type=jnp.float32)
        F8 = jnp.transpose(feat_ref[...][:, :8])
        e_row = jnp.where(s_row == 0., F8[0:1, :],
                          jnp.where(s_row == 1., F8[1:2, :], F8[2:3, :]))
        e_ref[...] = e_row.reshape(1, 1, BLK)
        x_ref[...] = F8[4:5, :].reshape(1, 1, BLK)
        y_ref[...] = F8[2:3, :].reshape(1, 1, BLK)
        z_ref[...] = F8[3:4, :].reshape(1, 1, BLK)

    row = jax.ShapeDtypeStruct((NTC, 1, BLK), jnp.float32)
    outs = pl.pallas_call(
        body,
        grid=(NTC,),
        in_specs=[
            pl.BlockSpec((BLK, CB), lambda i: (i, 0)),
            pl.BlockSpec((1, 1, BLK), lambda i: (i, 0, 0)),
            pl.BlockSpec((1, G), lambda i: (0, 0)),
        ],
        out_specs=[pl.BlockSpec((1, 1, BLK), lambda i: (i, 0, 0))] * 4,
        out_shape=[row] * 4,
        compiler_params=pltpu.CompilerParams(
            dimension_semantics=("arbitrary",)),
    )(features, batch3, state_row)

    return tuple(a.reshape(HT) for a in outs)


def _sc_part(features, state_flat, batch, LO, N):
    G = state_flat.shape[0]
    NSC = N - LO
    CPW = ((NSC + NW - 1) // NW + L - 1) // L * L
    C = 96
    NCHUNK = CPW // C
    NBUF = 5
    assert CPW % C == 0 and NCHUNK % NBUF == 0 and C % L == 0

    mesh = plsc.VectorSubcoreMesh(core_axis_name="c", subcore_axis_name="s")

    @functools.partial(
        pl.kernel,
        mesh=mesh,
        compiler_params=pltpu.CompilerParams(
            needs_layout_passes=False,
            skip_device_barrier=True,
            disable_bounds_checks=True,
            disable_semaphore_checks=True,
        ),
        out_type=tuple(
            jax.ShapeDtypeStruct((NSC,), jnp.float32) for _ in range(4)),
        scratch_types=(
            [pltpu.VMEM((C, CB), jnp.float32)] * NBUF
            + [
                pltpu.VMEM((CPW,), jnp.int32),
                pltpu.VMEM((G,), jnp.int32),
                pltpu.VMEM((CPW,), jnp.float32),
                pltpu.VMEM((CPW,), jnp.float32),
                pltpu.VMEM((CPW,), jnp.float32),
                pltpu.VMEM((CPW,), jnp.float32),
            ]
            + [pltpu.SemaphoreType.DMA] * (NBUF + 2)
        ),
    )
    def sc_kernel(feat_hbm, state_hbm, batch_hbm, e_hbm, x_hbm, y_hbm, z_hbm,
                  *scratch):
        fbufs = scratch[:NBUF]
        batch_v, state_v, e_v, x_v, y_v, z_v = scratch[NBUF:NBUF + 6]
        sems = scratch[NBUF + 6:NBUF + 6 + NBUF]
        sem_in, sem_out = scratch[NBUF + 6 + NBUF:]
        wid = lax.axis_index("s") * NC + lax.axis_index("c")
        base = LO + jnp.minimum(wid * CPW, NSC - CPW)

        def feat_dma(t, b):
            return pltpu.make_async_copy(
                feat_hbm.at[pl.ds(base + t * C, C), pl.ds(0, CB)],
                fbufs[b], sems[b])

        cp_state = pltpu.make_async_copy(state_hbm, state_v, sem_in)
        cp_batch = pltpu.make_async_copy(
            batch_hbm.at[pl.ds(base - LO, CPW)], batch_v, sem_in)
        cp_state.start()
        cp_batch.start()
        for t in range(NBUF - 1):
            feat_dma(t, t).start()
        cp_state.wait()
        cp_batch.wait()

        iota = lax.iota(jnp.int32, L)
        c2 = jnp.full((L,), 2, jnp.int32)
        c3 = jnp.full((L,), 3, jnp.int32)
        c4 = jnp.full((L,), 4, jnp.int32)

        def outer(k, carry):
            t0 = k * NBUF
            for b in range(NBUF):
                t = t0 + b
                nxt = t + NBUF - 1

                @pl.when(nxt < NCHUNK)
                def _(nxt=nxt, b=b):
                    feat_dma(nxt, (b + NBUF - 1) % NBUF).start()

                feat_dma(t, b).wait()
                fb = fbufs[b]

                def body(j, carry2, t=t, fb=fb):
                    n = iota + j * L
                    g = batch_v[pl.ds(t * C + j * L, L)]
                    s = plsc.load_gather(state_v, [g])
                    e = plsc.load_gather(fb, [n, s])
                    x = plsc.load_gather(fb, [n, c4])
                    y = plsc.load_gather(fb, [n, c2])
                    z = plsc.load_gather(fb, [n, c3])
                    o = pl.ds(t * C + j * L, L)
                    e_v[o] = e
                    x_v[o] = x
                    y_v[o] = y
                    z_v[o] = z
                    return carry2

                lax.fori_loop(0, C // L, body, 0)
            return carry

        lax.fori_loop(0, NCHUNK // NBUF, outer, 0)

        outs = [
            pltpu.make_async_copy(v, h.at[pl.ds(base - LO, CPW)], sem_out)
            for v, h in ((e_v, e_hbm), (x_v, x_hbm), (y_v, y_hbm), (z_v, z_hbm))
        ]
        for cp in outs:
            cp.start()
        for cp in outs:
            cp.wait()

    batch_tail = lax.slice_in_dim(batch, LO, N)
    return sc_kernel(features, state_flat, batch_tail)


def kernel(features, state, batch):
    N, D = features.shape
    G = state.shape[0]
    HT = NTC * BLK

    state_flat = state.reshape(G)
    state_row = state.reshape(1, G).astype(jnp.float32)
    batch3 = batch[:HT].reshape(NTC, 1, BLK)

    e_t, x_t, y_t, z_t = _tc_part(features, state_row, batch3, HT)
    e_s, x_s, y_s, z_s = _sc_part(features, state_flat, batch, HT, N)

    e = jnp.concatenate([e_t, e_s])
    x = jnp.concatenate([x_t, x_s])
    y = jnp.concatenate([y_t, y_s])
    z = jnp.concatenate([z_t, z_s])
    return e.reshape(N, 1), jnp.stack([x, y, z], axis=-1)


# speedup vs baseline: 9.9661x; 1.5627x over previous
import functools

import jax
import jax.numpy as jnp
from jax import lax
from jax.experimental import pallas as pl
from jax.experimental.pallas import tpu as pltpu
from jax.experimental.pallas import tpu_sc as plsc

L = 16
NC = 2
NS = 16
NW = NC * NS
CB = 128
BLK = 1024
NTC = 24


def _tc_part(features, state_row, batch3, HT):
    G = state_row.shape[1]

    def body(feat_ref, b_ref, s_ref, e_ref, x_ref, y_ref, z_ref):
        bc = b_ref[...].reshape(1, BLK)
        gids = lax.broadcasted_iota(jnp.int32, (G, BLK), 0)
        onehot_t = (bc == gids).astype(jnp.float32)
        s_row = lax.dot_general(s_ref[...], onehot_t,
                                (((1,), (0,)), ((), ())),
                                preferred_element_type=jnp.float32)
        F8 = jnp.transpose(feat_ref[...][:, :8])
        e_row = jnp.where(s_row == 0., F8[0:1, :],
                          jnp.where(s_row == 1., F8[1:2, :], F8[2:3, :]))
        e_ref[...] = e_row.reshape(1, 1, BLK)
        x_ref[...] = F8[4:5, :].reshape(1, 1, BLK)
        y_ref[...] = F8[2:3, :].reshape(1, 1, BLK)
        z_ref[...] = F8[3:4, :].reshape(1, 1, BLK)

    row = jax.ShapeDtypeStruct((NTC, 1, BLK), jnp.float32)
    outs = pl.pallas_call(
        body,
        grid=(NTC,),
        in_specs=[
            pl.BlockSpec((BLK, CB), lambda i: (i, 0)),
            pl.BlockSpec((1, 1, BLK), lambda i: (i, 0, 0)),
            pl.BlockSpec((1, G), lambda i: (0, 0)),
        ],
        out_specs=[pl.BlockSpec((1, 1, BLK), lambda i: (i, 0, 0))] * 4,
        out_shape=[row] * 4,
        compiler_params=pltpu.CompilerParams(
            dimension_semantics=("arbitrary",)),
    )(features, batch3, state_row)

    return tuple(a.reshape(HT) for a in outs)


def _sc_part(features, state_flat, batch, LO, N):
    G = state_flat.shape[0]
    NSC = N - LO
    C = 128
    NBUF = 5
    CPW = ((NSC + NW - 1) // NW + C * NBUF - 1) // (C * NBUF) * (C * NBUF)
    NCHUNK = CPW // C
    assert NW * CPW >= NSC and CPW <= NSC and C % L == 0

    mesh = plsc.VectorSubcoreMesh(core_axis_name="c", subcore_axis_name="s")

    @functools.partial(
        pl.kernel,
        mesh=mesh,
        compiler_params=pltpu.CompilerParams(
            needs_layout_passes=False,
            skip_device_barrier=True,
            disable_bounds_checks=True,
            disable_semaphore_checks=True,
        ),
        out_type=tuple(
            jax.ShapeDtypeStruct((NSC,), jnp.float32) for _ in range(4)),
        scratch_types=(
            [pltpu.VMEM((C, CB), jnp.float32)] * NBUF
            + [
                pltpu.VMEM((CPW,), jnp.int32),
                pltpu.VMEM((G,), jnp.int32),
                pltpu.VMEM((CPW,), jnp.float32),
                pltpu.VMEM((CPW,), jnp.float32),
                pltpu.VMEM((CPW,), jnp.float32),
                pltpu.VMEM((CPW,), jnp.float32),
            ]
            + [pltpu.SemaphoreType.DMA] * (NBUF + 2)
        ),
    )
    def sc_kernel(feat_hbm, state_hbm, batch_hbm, e_hbm, x_hbm, y_hbm, z_hbm,
                  *scratch):
        fbufs = scratch[:NBUF]
        batch_v, state_v, e_v, x_v, y_v, z_v = scratch[NBUF:NBUF + 6]
        sems = scratch[NBUF + 6:NBUF + 6 + NBUF]
        sem_in, sem_out = scratch[NBUF + 6 + NBUF:]
        wid = lax.axis_index("s") * NC + lax.axis_index("c")
        base = LO + jnp.minimum(wid * CPW, NSC - CPW)

        def feat_dma(t, b):
            return pltpu.make_async_copy(
                feat_hbm.at[pl.ds(base + t * C, C), pl.ds(0, CB)],
                fbufs[b], sems[b])

        cp_state = pltpu.make_async_copy(state_hbm, state_v, sem_in)
        cp_batch = pltpu.make_async_copy(
            batch_hbm.at[pl.ds(base - LO, CPW)], batch_v, sem_in)
        cp_state.start()
        cp_batch.start()
        for t in range(NBUF - 1):
            feat_dma(t, t).start()
        cp_state.wait()
        cp_batch.wait()

        iota = lax.iota(jnp.int32, L)
        c2 = jnp.full((L,), 2, jnp.int32)
        c3 = jnp.full((L,), 3, jnp.int32)
        c4 = jnp.full((L,), 4, jnp.int32)

        def outer(k, carry):
            t0 = k * NBUF
            for b in range(NBUF):
                t = t0 + b
                nxt = t + NBUF - 1

                @pl.when(nxt < NCHUNK)
                def _(nxt=nxt, b=b):
                    feat_dma(nxt, (b + NBUF - 1) % NBUF).start()

                feat_dma(t, b).wait()
                fb = fbufs[b]

                def body(j, carry2, t=t, fb=fb):
                    n = iota + j * L
                    g = batch_v[pl.ds(t * C + j * L, L)]
                    s = plsc.load_gather(state_v, [g])
                    e = plsc.load_gather(fb, [n, s])
                    x = plsc.load_gather(fb, [n, c4])
                    y = plsc.load_gather(fb, [n, c2])
                    z = plsc.load_gather(fb, [n, c3])
                    o = pl.ds(t * C + j * L, L)
                    e_v[o] = e
                    x_v[o] = x
                    y_v[o] = y
                    z_v[o] = z
                    return carry2

                lax.fori_loop(0, C // L, body, 0)
            return carry

        lax.fori_loop(0, NCHUNK // NBUF, outer, 0)

        outs = [
            pltpu.make_async_copy(v, h.at[pl.ds(base - LO, CPW)], sem_out)
            for v, h in ((e_v, e_hbm), (x_v, x_hbm), (y_v, y_hbm), (z_v, z_hbm))
        ]
        for cp in outs:
            cp.start()
        for cp in outs:
            cp.wait()

    batch_tail = lax.slice_in_dim(batch, LO, N)
    return sc_kernel(features, state_flat, batch_tail)


def kernel(features, state, batch):
    N, D = features.shape
    G = state.shape[0]
    HT = NTC * BLK

    state_flat = state.reshape(G)
    state_row = state.reshape(1, G).astype(jnp.float32)
    batch3 = batch[:HT].reshape(NTC, 1, BLK)

    e_t, x_t, y_t, z_t = _tc_part(features, state_row, batch3, HT)
    e_s, x_s, y_s, z_s = _sc_part(features, state_flat, batch, HT, N)

    e = jnp.concatenate([e_t, e_s])
    x = jnp.concatenate([x_t, x_s])
    y = jnp.concatenate([y_t, y_s])
    z = jnp.concatenate([z_t, z_s])
    return e.reshape(N, 1), jnp.stack([x, y, z], axis=-1)


# speedup vs baseline: 10.0510x; 1.0085x over previous
import functools

import jax
import jax.numpy as jnp
from jax import lax
from jax.experimental import pallas as pl
from jax.experimental.pallas import tpu as pltpu
from jax.experimental.pallas import tpu_sc as plsc

L = 16
NC = 2
NS = 16
NW = NC * NS
CB = 128


def kernel(features, state, batch):
    N, D = features.shape
    G = state.shape[0]
    CPW = ((N + NW - 1) // NW + L - 1) // L * L
    C = 112
    NCHUNK = CPW // C
    NBUF = 4
    assert NCHUNK % NBUF == 0

    state_flat = state.reshape(G)

    mesh = plsc.VectorSubcoreMesh(core_axis_name="c", subcore_axis_name="s")

    @functools.partial(
        pl.kernel,
        mesh=mesh,
        compiler_params=pltpu.CompilerParams(
            needs_layout_passes=False,
            skip_device_barrier=True,
            disable_bounds_checks=True,
            disable_semaphore_checks=True,
        ),
        out_type=(
            jax.ShapeDtypeStruct((N,), jnp.float32),
            jax.ShapeDtypeStruct((N,), jnp.float32),
            jax.ShapeDtypeStruct((N,), jnp.float32),
            jax.ShapeDtypeStruct((N,), jnp.float32),
        ),
        scratch_types=(
            [pltpu.VMEM((C, CB), jnp.float32)] * NBUF
            + [
                pltpu.VMEM((CPW,), jnp.int32),
                pltpu.VMEM((G,), jnp.int32),
                pltpu.VMEM((CPW,), jnp.float32),
                pltpu.VMEM((CPW,), jnp.float32),
                pltpu.VMEM((CPW,), jnp.float32),
                pltpu.VMEM((CPW,), jnp.float32),
            ]
            + [pltpu.SemaphoreType.DMA] * (NBUF + 2)
        ),
    )
    def sc_kernel(feat_hbm, state_hbm, batch_hbm, e_hbm, x_hbm, y_hbm, z_hbm,
                  *scratch):
        fbufs = scratch[:NBUF]
        batch_v, state_v, e_v, x_v, y_v, z_v = scratch[NBUF:NBUF + 6]
        sems = scratch[NBUF + 6:NBUF + 6 + NBUF]
        sem_in, sem_out = scratch[NBUF + 6 + NBUF:]
        wid = lax.axis_index("s") * NC + lax.axis_index("c")
        base = jnp.minimum(wid * CPW, N - CPW)

        def feat_dma(t, b):
            return pltpu.make_async_copy(
                feat_hbm.at[pl.ds(base + t * C, C), pl.ds(0, CB)],
                fbufs[b], sems[b])

        cp_state = pltpu.make_async_copy(state_hbm, state_v, sem_in)
        cp_batch = pltpu.make_async_copy(
            batch_hbm.at[pl.ds(base, CPW)], batch_v, sem_in)
        cp_state.start()
        cp_batch.start()
        for t in range(NBUF - 1):
            feat_dma(t, t).start()
        cp_state.wait()
        cp_batch.wait()

        iota = lax.iota(jnp.int32, L)
        c2 = jnp.full((L,), 2, jnp.int32)
        c3 = jnp.full((L,), 3, jnp.int32)
        c4 = jnp.full((L,), 4, jnp.int32)

        def outer(k, carry):
            t0 = k * NBUF
            for b in range(NBUF):
                t = t0 + b
                nxt = t + NBUF - 1

                @pl.when(nxt < NCHUNK)
                def _(nxt=nxt, b=b):
                    feat_dma(nxt, (b + NBUF - 1) % NBUF).start()

                feat_dma(t, b).wait()
                fb = fbufs[b]

                def body(j, carry2, t=t, fb=fb):
                    n = iota + j * L
                    g = batch_v[pl.ds(t * C + j * L, L)]
                    s = plsc.load_gather(state_v, [g])
                    e = plsc.load_gather(fb, [n, s])
                    x = plsc.load_gather(fb, [n, c4])
                    y = plsc.load_gather(fb, [n, c2])
                    z = plsc.load_gather(fb, [n, c3])
                    o = pl.ds(t * C + j * L, L)
                    e_v[o] = e
                    x_v[o] = x
                    y_v[o] = y
                    z_v[o] = z
                    return carry2

                lax.fori_loop(0, C // L, body, 0)
            return carry

        lax.fori_loop(0, NCHUNK // NBUF, outer, 0)

        outs = [
            pltpu.make_async_copy(v, h.at[pl.ds(base, CPW)], sem_out)
            for v, h in ((e_v, e_hbm), (x_v, x_hbm), (y_v, y_hbm), (z_v, z_hbm))
        ]
        for cp in outs:
            cp.start()
        for cp in outs:
            cp.wait()

    e, x, y, z = sc_kernel(features, state_flat, batch)
    return e.reshape(N, 1), jnp.stack([x, y, z], axis=-1)
